# baseline (speedup 1.0000x reference)
import functools

import jax
import jax.numpy as jnp
from jax import lax
from jax.experimental import pallas as pl
from jax.experimental.pallas import tpu as pltpu
from jax.experimental.pallas import tpu_sc as plsc

N = 10000
E = 320000
D = 128

NC = 2
NS = 16
NW = NC * NS
EPW = E // NW
CH = 80
NCHUNK = EPW // CH

@functools.cache
def _mesh():
    return plsc.VectorSubcoreMesh(core_axis_name="c", subcore_axis_name="s",
                                  num_cores=NC, num_subcores=NS)


def _ln(v, scale, bias):
    mu = jnp.mean(v, axis=-1, keepdims=True)
    var = jnp.mean((v - mu) ** 2, axis=-1, keepdims=True)
    return (v - mu) * lax.rsqrt(var + 1e-5) * scale + bias


def _mlp_body(x_ref, w1_ref, b1_ref, w2_ref, b2_ref, s_ref, t_ref, o_ref):
    u = jnp.maximum(
        jnp.dot(x_ref[...], w1_ref[...], preferred_element_type=jnp.float32)
        + b1_ref[...], 0.0)
    v = jnp.dot(u, w2_ref[...], preferred_element_type=jnp.float32) + b2_ref[...]
    o_ref[...] = _ln(v, s_ref[...], t_ref[...])


def _row2(a):
    return a.reshape(1, -1)


def _mlp(x, p, block_rows):
    (w1, b1), (w2, b2) = p["layers"]
    rows = x.shape[0]
    grid = (rows // block_rows,)
    full = lambda i: (0, 0)
    return pl.pallas_call(
        _mlp_body,
        grid=grid,
        in_specs=[
            pl.BlockSpec((block_rows, x.shape[1]), lambda i: (i, 0)),
            pl.BlockSpec(w1.shape, full),
            pl.BlockSpec((1, D), full),
            pl.BlockSpec(w2.shape, full),
            pl.BlockSpec((1, D), full),
            pl.BlockSpec((1, D), full),
            pl.BlockSpec((1, D), full),
        ],
        out_specs=pl.BlockSpec((block_rows, D), lambda i: (i, 0)),
        out_shape=jax.ShapeDtypeStruct((rows, D), jnp.float32),
    )(x, w1, _row2(b1), w2, _row2(b2), _row2(p["ln_scale"]), _row2(p["ln_bias"]))


def _edge_step_body(e_ref, g_ref, wc_ref, b1_ref,
                    w2_ref, b2_ref, s_ref, t_ref, enew_ref, eout_ref):
    e = e_ref[...]
    pre = (g_ref[...]
           + jnp.dot(e, wc_ref[...], preferred_element_type=jnp.float32)
           + b1_ref[...])
    u = jnp.maximum(pre, 0.0)
    v = jnp.dot(u, w2_ref[...], preferred_element_type=jnp.float32) + b2_ref[...]
    v = _ln(v, s_ref[...], t_ref[...])
    enew_ref[...] = v
    eout_ref[...] = e + v


def _edge_step(e, g, p, block_rows=1600):
    (w1, b1), (w2, b2) = p["layers"]
    wc = w1[2 * D:3 * D]
    grid = (E // block_rows,)
    full = lambda i: (0, 0)
    blk = lambda i: (i, 0)
    return pl.pallas_call(
        _edge_step_body,
        grid=grid,
        in_specs=[
            pl.BlockSpec((block_rows, D), blk),
            pl.BlockSpec((block_rows, D), blk),
            pl.BlockSpec((D, D), full),
            pl.BlockSpec((1, D), full),
            pl.BlockSpec((D, D), full),
            pl.BlockSpec((1, D), full),
            pl.BlockSpec((1, D), full),
            pl.BlockSpec((1, D), full),
        ],
        out_specs=(pl.BlockSpec((block_rows, D), blk),
                   pl.BlockSpec((block_rows, D), blk)),
        out_shape=(jax.ShapeDtypeStruct((E, D), jnp.float32),
                   jax.ShapeDtypeStruct((E, D), jnp.float32)),
    )(e, g, wc, _row2(b1), w2, _row2(b2),
      _row2(p["ln_scale"]), _row2(p["ln_bias"]))


def _node_step_body(h_ref, a0_ref, a1_ref, wh_ref, wg_ref, b1_ref, w2_ref,
                    b2_ref, s_ref, t_ref, o_ref):
    h = h_ref[...]
    agg = a0_ref[...] + a1_ref[...]
    u = jnp.maximum(
        jnp.dot(h, wh_ref[...], preferred_element_type=jnp.float32)
        + jnp.dot(agg, wg_ref[...], preferred_element_type=jnp.float32)
        + b1_ref[...], 0.0)
    v = jnp.dot(u, w2_ref[...], preferred_element_type=jnp.float32) + b2_ref[...]
    o_ref[...] = h + _ln(v, s_ref[...], t_ref[...])


def _node_step_tables_body(h_ref, a0_ref, a1_ref, wh_ref, wg_ref, b1_ref,
                           w2_ref, b2_ref, s_ref, t_ref, wa_ref, wb_ref,
                           o_ref, ha_ref, hb_ref):
    h = h_ref[...]
    agg = a0_ref[...] + a1_ref[...]
    u = jnp.maximum(
        jnp.dot(h, wh_ref[...], preferred_element_type=jnp.float32)
        + jnp.dot(agg, wg_ref[...], preferred_element_type=jnp.float32)
        + b1_ref[...], 0.0)
    v = jnp.dot(u, w2_ref[...], preferred_element_type=jnp.float32) + b2_ref[...]
    ho = h + _ln(v, s_ref[...], t_ref[...])
    o_ref[...] = ho
    ha_ref[...] = jnp.dot(ho, wa_ref[...], preferred_element_type=jnp.float32)
    hb_ref[...] = jnp.dot(ho, wb_ref[...], preferred_element_type=jnp.float32)


def _node_step(h, a0, a1, p, wa=None, wb=None, tables=False, block_rows=2000):
    (w1, b1), (w2, b2) = p["layers"]
    wh, wg = w1[0:D], w1[D:2 * D]
    grid = (N // block_rows,)
    full = lambda i: (0, 0)
    blk = lambda i: (i, 0)
    specs = [
        pl.BlockSpec((block_rows, D), blk),
        pl.BlockSpec((block_rows, D), blk),
        pl.BlockSpec((block_rows, D), blk),
        pl.BlockSpec((D, D), full),
        pl.BlockSpec((D, D), full),
        pl.BlockSpec((1, D), full),
        pl.BlockSpec((D, D), full),
        pl.BlockSpec((1, D), full),
        pl.BlockSpec((1, D), full),
        pl.BlockSpec((1, D), full),
    ]
    args = [h, a0, a1, wh, wg, _row2(b1), w2, _row2(b2),
            _row2(p["ln_scale"]), _row2(p["ln_bias"])]
    if not tables:
        return pl.pallas_call(
            _node_step_body,
            grid=grid,
            in_specs=specs,
            out_specs=pl.BlockSpec((block_rows, D), blk),
            out_shape=jax.ShapeDtypeStruct((N, D), jnp.float32),
        )(*args)
    specs += [pl.BlockSpec((D, D), full), pl.BlockSpec((D, D), full)]
    args += [wa, wb]
    return pl.pallas_call(
        _node_step_tables_body,
        grid=grid,
        in_specs=specs,
        out_specs=(pl.BlockSpec((block_rows, D), blk),) * 3,
        out_shape=(jax.ShapeDtypeStruct((N, D), jnp.float32),) * 3,
    )(*args)


def _tables_body(h_ref, wa_ref, wb_ref, ha_ref, hb_ref):
    h = h_ref[...]
    ha_ref[...] = jnp.dot(h, wa_ref[...], preferred_element_type=jnp.float32)
    hb_ref[...] = jnp.dot(h, wb_ref[...], preferred_element_type=jnp.float32)


def _tables(h, wa, wb, block_rows=2000):
    grid = (N // block_rows,)
    full = lambda i: (0, 0)
    blk = lambda i: (i, 0)
    return pl.pallas_call(
        _tables_body,
        grid=grid,
        in_specs=[
            pl.BlockSpec((block_rows, D), blk),
            pl.BlockSpec((D, D), full),
            pl.BlockSpec((D, D), full),
        ],
        out_specs=(pl.BlockSpec((block_rows, D), blk),) * 2,
        out_shape=(jax.ShapeDtypeStruct((N, D), jnp.float32),) * 2,
    )(h, wa, wb)


def _gather_body(ha_hbm, hb_hbm, src_hbm, dst_hbm, g_hbm,
                 sidx, didx,
                 ba0, ba1, ba2, bb0, bb1, bb2, wb0, wb1, wb2,
                 sa0, sa1, sa2, sb0, sb1, sb2, sw0, sw1, sw2):
    c = lax.axis_index("c")
    s = lax.axis_index("s")
    w = c * NS + s
    base = w * EPW
    pltpu.sync_copy(src_hbm.at[w], sidx)
    pltpu.sync_copy(dst_hbm.at[w], didx)

    slots = ((ba0, bb0, wb0, sa0, sb0, sw0),
             (ba1, bb1, wb1, sa1, sb1, sw1),
             (ba2, bb2, wb2, sa2, sb2, sw2))

    def out_ref(j):
        return g_hbm.at[pl.ds(pl.multiple_of(base + j * CH, CH), CH)]

    def start_g(j, t):
        ba, bb, _, sa, sb, _ = slots[t]
        pltpu.async_copy(ha_hbm.at[sidx.at[j]], ba, sa)
        pltpu.async_copy(hb_hbm.at[didx.at[j]], bb, sb)

    def visit(j, t, *, first, last):
        ba, bb, wb, sa, sb, sw = slots[t]
        pltpu.make_async_copy(ha_hbm.at[sidx.at[j]], ba, sa).wait()
        pltpu.make_async_copy(hb_hbm.at[didx.at[j]], bb, sb).wait()
        if not first:
            pltpu.make_async_copy(wb, out_ref(j - 3), sw).wait()

        def addrows(r4, carry):
            for r0 in range(4):
                r = r4 * 4 + r0
                for k in range(D // 16):
                    sl = pl.ds(k * 16, 16)
                    wb[r, sl] = ba[r, sl] + bb[r, sl]
            return carry

        lax.fori_loop(0, CH // 4, addrows, 0)
        if not last:
            start_g(j + 3, t)
        pltpu.async_copy(wb, out_ref(j), sw)

    start_g(0, 0)
    start_g(1, 1)
    start_g(2, 2)
    visit(0, 0, first=True, last=False)
    visit(1, 1, first=True, last=False)
    visit(2, 2, first=True, last=False)

    def body(i, carry):
        v0 = 3 * i + 3
        visit(v0, 0, first=False, last=False)
        visit(v0 + 1, 1, first=False, last=False)
        visit(v0 + 2, 2, first=False, last=False)
        return carry

    lax.fori_loop(0, 39, body, 0)
    visit(120, 0, first=False, last=False)
    visit(121, 1, first=False, last=False)
    visit(122, 2, first=False, last=True)
    visit(123, 0, first=False, last=True)
    visit(124, 1, first=False, last=True)
    for j, t in ((122, 2), (123, 0), (124, 1)):
        _, _, wb, _, _, sw = slots[t]
        pltpu.make_async_copy(wb, out_ref(j), sw).wait()


@functools.cache
def _sc_gather_kernel():
    return pl.kernel(
        _gather_body,
        out_type=jax.ShapeDtypeStruct((E, D), jnp.float32),
        mesh=_mesh(),
        scratch_types=(
            [pltpu.VMEM((NCHUNK, CH), jnp.int32)] * 2
            + [pltpu.VMEM((CH, D), jnp.float32)] * 9
            + [pltpu.SemaphoreType.DMA] * 9
        ),
    )


def _sc_gather(ha, hb, src_r, dst_r):
    return _sc_gather_kernel()(ha, hb, src_r, dst_r)


ZROWS = 632
ZLAST = N - (NS - 1) * ZROWS
ZB = 8


def _scatter_body(enew_hbm, dst_hbm, agg_hbm, didx,
                  b0, b1, b2, zbuf, acc,
                  sf0, sf1, sf2, sa0, sa1, sa2, semz):
    c = lax.axis_index("c")
    s = lax.axis_index("s")
    w = c * NS + s

    z16 = jnp.zeros((16,), jnp.float32)
    for i in range(ZB):
        for k in range(D // 16):
            zbuf[i, pl.ds(k * 16, 16)] = z16

    my_base = pl.multiple_of(s * ZROWS, ZROWS)
    my_rows = jnp.where(s == NS - 1, ZLAST, ZROWS)

    def zrow(r, carry):
        pltpu.async_copy(
            zbuf, acc.at[pl.ds(pl.multiple_of(my_base + r * ZB, ZB), ZB)], semz)
        return carry

    def zdrain(r, carry):
        pltpu.make_async_copy(
            zbuf, acc.at[pl.ds(pl.multiple_of(my_base + r * ZB, ZB), ZB)],
            semz).wait()
        return carry

    nz = my_rows // ZB
    lax.fori_loop(0, nz, zrow, 0)
    lax.fori_loop(0, nz, zdrain, 0)
    plsc.subcore_barrier()

    pltpu.sync_copy(dst_hbm.at[w], didx)

    slots = ((b0, sf0, sa0), (b1, sf1, sa1), (b2, sf2, sa2))

    def in_ref(j):
        return enew_hbm.at[pl.ds(pl.multiple_of(w * EPW + j * CH, CH), CH)]

    def fetch(j, t):
        b, sf, _ = slots[t]
        pltpu.async_copy(in_ref(j), b, sf)

    def wait_add(j, t):
        b, _, sa = slots[t]
        pltpu.make_async_copy(b, acc.at[didx.at[j]], sa).wait()

    def visit(j, t, *, refill):
        b, sf, sa = slots[t]
        pltpu.make_async_copy(in_ref(j), b, sf).wait()
        pltpu.async_copy(b, acc.at[didx.at[j]], sa, add=True)
        if refill:
            u = (t + 1) % 3
            wait_add(j - 2, u)
            fetch(j + 1, u)

    fetch(0, 0)
    fetch(1, 1)
    fetch(2, 2)
    visit(0, 0, refill=False)
    visit(1, 1, refill=False)
    visit(2, 2, refill=True)

    def body(i, carry):
        v0 = 3 * i + 3
        visit(v0, 0, refill=True)
        visit(v0 + 1, 1, refill=True)
        visit(v0 + 2, 2, refill=True)
        return carry

    lax.fori_loop(0, 39, body, 0)
    visit(120, 0, refill=True)
    visit(121, 1, refill=True)
    visit(122, 2, refill=True)
    visit(123, 0, refill=True)
    visit(124, 1, refill=False)
    wait_add(122, 2)
    wait_add(123, 0)
    wait_add(124, 1)
    plsc.subcore_barrier()

    @pl.when(s < NS - 1)
    def _():
        pltpu.async_copy(acc.at[pl.ds(my_base, ZROWS)],
                         agg_hbm.at[c, pl.ds(my_base, ZROWS)], semz).wait()

    @pl.when(s == NS - 1)
    def _():
        pltpu.async_copy(acc.at[pl.ds((NS - 1) * ZROWS, ZLAST)],
                         agg_hbm.at[c, pl.ds((NS - 1) * ZROWS, ZLAST)],
                         semz).wait()


@functools.cache
def _sc_scatter_kernel():
    return pl.kernel(
        _scatter_body,
        out_type=jax.ShapeDtypeStruct((NC, N, D), jnp.float32),
        mesh=_mesh(),
        scratch_types=(
            [pltpu.VMEM((NCHUNK, CH), jnp.int32)]
            + [pltpu.VMEM((CH, D), jnp.float32)] * 3
            + [pltpu.VMEM((ZB, D), jnp.float32)]
            + [pltpu.VMEM_SHARED((N, D), jnp.float32)]
            + [pltpu.SemaphoreType.DMA] * 7
        ),
    )


def _sc_scatter(e_new, dst_r):
    return _sc_scatter_kernel()(e_new, dst_r)


def kernel(x, edge_index, edge_features, params):
    src_r = edge_index[0].reshape(NW, NCHUNK, CH)
    dst_r = edge_index[1].reshape(NW, NCHUNK, CH)

    h = _mlp(x, params["enc_node"], block_rows=2000)
    e = _mlp(edge_features, params["enc_edge"], block_rows=1600)

    wa0 = params["proc"][0]["edge"]["layers"][0][0][0:D]
    wb0 = params["proc"][0]["edge"]["layers"][0][0][D:2 * D]
    ha, hb = _tables(h, wa0, wb0)
    for i, p in enumerate(params["proc"]):
        g = _sc_gather(ha, hb, src_r, dst_r)
        e_new, e = _edge_step(e, g, p["edge"])
        agg = _sc_scatter(e_new, dst_r)
        if i + 1 < len(params["proc"]):
            wan = params["proc"][i + 1]["edge"]["layers"][0][0][0:D]
            wbn = params["proc"][i + 1]["edge"]["layers"][0][0][D:2 * D]
            h, ha, hb = _node_step(h, agg[0], agg[1], p["node"],
                                   wan, wbn, tables=True)
        else:
            h = _node_step(h, agg[0], agg[1], p["node"])

    return (_mlp(h, params["dec_node"], block_rows=2000),
            _mlp(e, params["dec_edge"], block_rows=1600))


# speedup vs baseline: 1.0797x; 1.0797x over previous
import functools

import jax
import jax.numpy as jnp
from jax import lax
from jax.experimental import pallas as pl
from jax.experimental.pallas import tpu as pltpu
from jax.experimental.pallas import tpu_sc as plsc

N = 10000
E = 320000
D = 128

NC = 2
NS = 16
NW = NC * NS
EPW = E // NW
CH = 80
NCHUNK = EPW // CH

@functools.cache
def _mesh():
    return plsc.VectorSubcoreMesh(core_axis_name="c", subcore_axis_name="s",
                                  num_cores=NC, num_subcores=NS)


def _ln(v, scale, bias):
    mu = jnp.mean(v, axis=-1, keepdims=True)
    var = jnp.mean((v - mu) ** 2, axis=-1, keepdims=True)
    return (v - mu) * lax.rsqrt(var + 1e-5) * scale + bias


def _mlp_body(x_ref, w1_ref, b1_ref, w2_ref, b2_ref, s_ref, t_ref, o_ref):
    u = jnp.maximum(
        jnp.dot(x_ref[...], w1_ref[...], preferred_element_type=jnp.float32)
        + b1_ref[...], 0.0)
    v = jnp.dot(u, w2_ref[...], preferred_element_type=jnp.float32) + b2_ref[...]
    o_ref[...] = _ln(v, s_ref[...], t_ref[...])


def _row2(a):
    return a.reshape(1, -1)


def _mlp(x, p, block_rows):
    (w1, b1), (w2, b2) = p["layers"]
    rows = x.shape[0]
    grid = (rows // block_rows,)
    full = lambda i: (0, 0)
    return pl.pallas_call(
        _mlp_body,
        grid=grid,
        in_specs=[
            pl.BlockSpec((block_rows, x.shape[1]), lambda i: (i, 0)),
            pl.BlockSpec(w1.shape, full),
            pl.BlockSpec((1, D), full),
            pl.BlockSpec(w2.shape, full),
            pl.BlockSpec((1, D), full),
            pl.BlockSpec((1, D), full),
            pl.BlockSpec((1, D), full),
        ],
        out_specs=pl.BlockSpec((block_rows, D), lambda i: (i, 0)),
        out_shape=jax.ShapeDtypeStruct((rows, D), jnp.float32),
    )(x, w1, _row2(b1), w2, _row2(b2), _row2(p["ln_scale"]), _row2(p["ln_bias"]))


def _edge_step_body(e_ref, g_ref, wc_ref, b1_ref,
                    w2_ref, b2_ref, s_ref, t_ref, enew_ref, eout_ref):
    e = e_ref[...]
    pre = (g_ref[...]
           + jnp.dot(e, wc_ref[...], preferred_element_type=jnp.float32)
           + b1_ref[...])
    u = jnp.maximum(pre, 0.0)
    v = jnp.dot(u, w2_ref[...], preferred_element_type=jnp.float32) + b2_ref[...]
    v = _ln(v, s_ref[...], t_ref[...])
    enew_ref[...] = v
    eout_ref[...] = e + v


def _edge_step(e, g, p, block_rows=1600):
    (w1, b1), (w2, b2) = p["layers"]
    wc = w1[2 * D:3 * D]
    grid = (E // block_rows,)
    full = lambda i: (0, 0)
    blk = lambda i: (i, 0)
    return pl.pallas_call(
        _edge_step_body,
        grid=grid,
        in_specs=[
            pl.BlockSpec((block_rows, D), blk),
            pl.BlockSpec((block_rows, D), blk),
            pl.BlockSpec((D, D), full),
            pl.BlockSpec((1, D), full),
            pl.BlockSpec((D, D), full),
            pl.BlockSpec((1, D), full),
            pl.BlockSpec((1, D), full),
            pl.BlockSpec((1, D), full),
        ],
        out_specs=(pl.BlockSpec((block_rows, D), blk),
                   pl.BlockSpec((block_rows, D), blk)),
        out_shape=(jax.ShapeDtypeStruct((E, D), jnp.float32),
                   jax.ShapeDtypeStruct((E, D), jnp.float32)),
    )(e, g, wc, _row2(b1), w2, _row2(b2),
      _row2(p["ln_scale"]), _row2(p["ln_bias"]))


def _node_step_body(h_ref, a0_ref, a1_ref, wh_ref, wg_ref, b1_ref, w2_ref,
                    b2_ref, s_ref, t_ref, o_ref):
    h = h_ref[...]
    agg = a0_ref[...] + a1_ref[...]
    u = jnp.maximum(
        jnp.dot(h, wh_ref[...], preferred_element_type=jnp.float32)
        + jnp.dot(agg, wg_ref[...], preferred_element_type=jnp.float32)
        + b1_ref[...], 0.0)
    v = jnp.dot(u, w2_ref[...], preferred_element_type=jnp.float32) + b2_ref[...]
    o_ref[...] = h + _ln(v, s_ref[...], t_ref[...])


def _node_step_tables_body(h_ref, a0_ref, a1_ref, wh_ref, wg_ref, b1_ref,
                           w2_ref, b2_ref, s_ref, t_ref, wa_ref, wb_ref,
                           o_ref, ha_ref, hb_ref):
    h = h_ref[...]
    agg = a0_ref[...] + a1_ref[...]
    u = jnp.maximum(
        jnp.dot(h, wh_ref[...], preferred_element_type=jnp.float32)
        + jnp.dot(agg, wg_ref[...], preferred_element_type=jnp.float32)
        + b1_ref[...], 0.0)
    v = jnp.dot(u, w2_ref[...], preferred_element_type=jnp.float32) + b2_ref[...]
    ho = h + _ln(v, s_ref[...], t_ref[...])
    o_ref[...] = ho
    ha_ref[...] = jnp.dot(ho, wa_ref[...], preferred_element_type=jnp.float32)
    hb_ref[...] = jnp.dot(ho, wb_ref[...], preferred_element_type=jnp.float32)


def _node_step(h, a0, a1, p, wa=None, wb=None, tables=False, block_rows=2000):
    (w1, b1), (w2, b2) = p["layers"]
    wh, wg = w1[0:D], w1[D:2 * D]
    grid = (N // block_rows,)
    full = lambda i: (0, 0)
    blk = lambda i: (i, 0)
    specs = [
        pl.BlockSpec((block_rows, D), blk),
        pl.BlockSpec((block_rows, D), blk),
        pl.BlockSpec((block_rows, D), blk),
        pl.BlockSpec((D, D), full),
        pl.BlockSpec((D, D), full),
        pl.BlockSpec((1, D), full),
        pl.BlockSpec((D, D), full),
        pl.BlockSpec((1, D), full),
        pl.BlockSpec((1, D), full),
        pl.BlockSpec((1, D), full),
    ]
    args = [h, a0, a1, wh, wg, _row2(b1), w2, _row2(b2),
            _row2(p["ln_scale"]), _row2(p["ln_bias"])]
    if not tables:
        return pl.pallas_call(
            _node_step_body,
            grid=grid,
            in_specs=specs,
            out_specs=pl.BlockSpec((block_rows, D), blk),
            out_shape=jax.ShapeDtypeStruct((N, D), jnp.float32),
        )(*args)
    specs += [pl.BlockSpec((D, D), full), pl.BlockSpec((D, D), full)]
    args += [wa, wb]
    return pl.pallas_call(
        _node_step_tables_body,
        grid=grid,
        in_specs=specs,
        out_specs=(pl.BlockSpec((block_rows, D), blk),) * 3,
        out_shape=(jax.ShapeDtypeStruct((N, D), jnp.float32),) * 3,
    )(*args)


def _tables_body(h_ref, wa_ref, wb_ref, ha_ref, hb_ref):
    h = h_ref[...]
    ha_ref[...] = jnp.dot(h, wa_ref[...], preferred_element_type=jnp.float32)
    hb_ref[...] = jnp.dot(h, wb_ref[...], preferred_element_type=jnp.float32)


def _tables(h, wa, wb, block_rows=2000):
    grid = (N // block_rows,)
    full = lambda i: (0, 0)
    blk = lambda i: (i, 0)
    return pl.pallas_call(
        _tables_body,
        grid=grid,
        in_specs=[
            pl.BlockSpec((block_rows, D), blk),
            pl.BlockSpec((D, D), full),
            pl.BlockSpec((D, D), full),
        ],
        out_specs=(pl.BlockSpec((block_rows, D), blk),) * 2,
        out_shape=(jax.ShapeDtypeStruct((N, D), jnp.float32),) * 2,
    )(h, wa, wb)


def _gather_body(ha_hbm, hb_hbm, src_hbm, dst_hbm, g_hbm,
                 sidx, didx,
                 ba0, ba1, ba2, bb0, bb1, bb2, wb0, wb1, wb2,
                 sa0, sa1, sa2, sb0, sb1, sb2, sw0, sw1, sw2):
    c = lax.axis_index("c")
    s = lax.axis_index("s")
    w = c * NS + s
    base = w * EPW
    pltpu.sync_copy(src_hbm.at[w], sidx)
    pltpu.sync_copy(dst_hbm.at[w], didx)

    slots = ((ba0, bb0, wb0, sa0, sb0, sw0),
             (ba1, bb1, wb1, sa1, sb1, sw1),
             (ba2, bb2, wb2, sa2, sb2, sw2))

    def out_ref(j):
        return g_hbm.at[pl.ds(pl.multiple_of(base + j * CH, CH), CH)]

    def start_g(j, t):
        ba, bb, _, sa, sb, _ = slots[t]
        pltpu.async_copy(ha_hbm.at[sidx.at[j]], ba, sa)
        pltpu.async_copy(hb_hbm.at[didx.at[j]], bb, sb)

    def visit(j, t, *, first, last):
        ba, bb, wb, sa, sb, sw = slots[t]
        pltpu.make_async_copy(ha_hbm.at[sidx.at[j]], ba, sa).wait()
        pltpu.make_async_copy(hb_hbm.at[didx.at[j]], bb, sb).wait()
        if not first:
            pltpu.make_async_copy(wb, out_ref(j - 3), sw).wait()

        def addrows(r4, carry):
            for r0 in range(4):
                r = r4 * 4 + r0
                for k in range(D // 16):
                    sl = pl.ds(k * 16, 16)
                    wb[r, sl] = ba[r, sl] + bb[r, sl]
            return carry

        lax.fori_loop(0, CH // 4, addrows, 0)
        if not last:
            start_g(j + 3, t)
        pltpu.async_copy(wb, out_ref(j), sw)

    start_g(0, 0)
    start_g(1, 1)
    start_g(2, 2)
    visit(0, 0, first=True, last=False)
    visit(1, 1, first=True, last=False)
    visit(2, 2, first=True, last=False)

    def body(i, carry):
        v0 = 3 * i + 3
        visit(v0, 0, first=False, last=False)
        visit(v0 + 1, 1, first=False, last=False)
        visit(v0 + 2, 2, first=False, last=False)
        return carry

    lax.fori_loop(0, 39, body, 0)
    visit(120, 0, first=False, last=False)
    visit(121, 1, first=False, last=False)
    visit(122, 2, first=False, last=True)
    visit(123, 0, first=False, last=True)
    visit(124, 1, first=False, last=True)
    for j, t in ((122, 2), (123, 0), (124, 1)):
        _, _, wb, _, _, sw = slots[t]
        pltpu.make_async_copy(wb, out_ref(j), sw).wait()


@functools.cache
def _sc_gather_kernel():
    return pl.kernel(
        _gather_body,
        out_type=jax.ShapeDtypeStruct((E, D), jnp.float32),
        mesh=_mesh(),
        scratch_types=(
            [pltpu.VMEM((NCHUNK, CH), jnp.int32)] * 2
            + [pltpu.VMEM((CH, D), jnp.float32)] * 9
            + [pltpu.SemaphoreType.DMA] * 9
        ),
    )


def _sc_gather(ha, hb, src_r, dst_r):
    return _sc_gather_kernel()(ha, hb, src_r, dst_r)


ZROWS = 632
ZLAST = N - (NS - 1) * ZROWS
ZB = 8


def _scatter_body(enew_hbm, dst_hbm, agg_hbm, didx,
                  b0, b1, b2, zbuf, acc,
                  sf0, sf1, sf2, sa0, sa1, sa2, semz):
    c = lax.axis_index("c")
    s = lax.axis_index("s")
    w = c * NS + s

    z16 = jnp.zeros((16,), jnp.float32)
    for i in range(ZB):
        for k in range(D // 16):
            zbuf[i, pl.ds(k * 16, 16)] = z16

    my_base = pl.multiple_of(s * ZROWS, ZROWS)
    my_rows = jnp.where(s == NS - 1, ZLAST, ZROWS)

    def zrow(r, carry):
        pltpu.async_copy(
            zbuf, acc.at[pl.ds(pl.multiple_of(my_base + r * ZB, ZB), ZB)], semz)
        return carry

    def zdrain(r, carry):
        pltpu.make_async_copy(
            zbuf, acc.at[pl.ds(pl.multiple_of(my_base + r * ZB, ZB), ZB)],
            semz).wait()
        return carry

    nz = my_rows // ZB
    lax.fori_loop(0, nz, zrow, 0)
    lax.fori_loop(0, nz, zdrain, 0)
    plsc.subcore_barrier()

    pltpu.sync_copy(dst_hbm.at[w], didx)

    slots = ((b0, sf0, sa0), (b1, sf1, sa1), (b2, sf2, sa2))

    def in_ref(j):
        return enew_hbm.at[pl.ds(pl.multiple_of(w * EPW + j * CH, CH), CH)]

    def fetch(j, t):
        b, sf, _ = slots[t]
        pltpu.async_copy(in_ref(j), b, sf)

    def wait_add(j, t):
        b, _, sa = slots[t]
        pltpu.make_async_copy(b, acc.at[didx.at[j]], sa).wait()

    def sync_add(j, t):
        b, sf, sa = slots[t]
        pltpu.make_async_copy(in_ref(j), b, sf).wait()
        pltpu.sync_copy(b, acc.at[didx.at[j]], add=True)

    fetch(0, 0)
    fetch(1, 1)
    fetch(2, 2)

    def body(i, carry):
        v0 = 3 * i
        sync_add(v0, 0)
        fetch(v0 + 3, 0)
        sync_add(v0 + 1, 1)
        fetch(v0 + 4, 1)
        sync_add(v0 + 2, 2)
        fetch(v0 + 5, 2)
        return carry

    lax.fori_loop(0, (NCHUNK - 5) // 3, body, 0)
    sync_add(120, 0)
    fetch(123, 0)
    sync_add(121, 1)
    fetch(124, 1)
    sync_add(122, 2)
    sync_add(123, 0)
    sync_add(124, 1)
    plsc.subcore_barrier()

    @pl.when(s < NS - 1)
    def _():
        pltpu.async_copy(acc.at[pl.ds(my_base, ZROWS)],
                         agg_hbm.at[c, pl.ds(my_base, ZROWS)], semz).wait()

    @pl.when(s == NS - 1)
    def _():
        pltpu.async_copy(acc.at[pl.ds((NS - 1) * ZROWS, ZLAST)],
                         agg_hbm.at[c, pl.ds((NS - 1) * ZROWS, ZLAST)],
                         semz).wait()


@functools.cache
def _sc_scatter_kernel():
    return pl.kernel(
        _scatter_body,
        out_type=jax.ShapeDtypeStruct((NC, N, D), jnp.float32),
        mesh=_mesh(),
        scratch_types=(
            [pltpu.VMEM((NCHUNK, CH), jnp.int32)]
            + [pltpu.VMEM((CH, D), jnp.float32)] * 3
            + [pltpu.VMEM((ZB, D), jnp.float32)]
            + [pltpu.VMEM_SHARED((N, D), jnp.float32)]
            + [pltpu.SemaphoreType.DMA] * 7
        ),
    )


def _sc_scatter(e_new, dst_r):
    return _sc_scatter_kernel()(e_new, dst_r)


def kernel(x, edge_index, edge_features, params):
    src_r = edge_index[0].reshape(NW, NCHUNK, CH)
    dst_r = edge_index[1].reshape(NW, NCHUNK, CH)

    h = _mlp(x, params["enc_node"], block_rows=2000)
    e = _mlp(edge_features, params["enc_edge"], block_rows=1600)

    wa0 = params["proc"][0]["edge"]["layers"][0][0][0:D]
    wb0 = params["proc"][0]["edge"]["layers"][0][0][D:2 * D]
    ha, hb = _tables(h, wa0, wb0)
    for i, p in enumerate(params["proc"]):
        g = _sc_gather(ha, hb, src_r, dst_r)
        e_new, e = _edge_step(e, g, p["edge"])
        agg = _sc_scatter(e_new, dst_r)
        if i + 1 < len(params["proc"]):
            wan = params["proc"][i + 1]["edge"]["layers"][0][0][0:D]
            wbn = params["proc"][i + 1]["edge"]["layers"][0][0][D:2 * D]
            h, ha, hb = _node_step(h, agg[0], agg[1], p["node"],
                                   wan, wbn, tables=True)
        else:
            h = _node_step(h, agg[0], agg[1], p["node"])

    return (_mlp(h, params["dec_node"], block_rows=2000),
            _mlp(e, params["dec_edge"], block_rows=1600))


# speedup vs baseline: 1.1188x; 1.0362x over previous
import functools

import jax
import jax.numpy as jnp
from jax import lax
from jax.experimental import pallas as pl
from jax.experimental.pallas import tpu as pltpu
from jax.experimental.pallas import tpu_sc as plsc

N = 10000
E = 320000
D = 128

NC = 2
NS = 16
NW = NC * NS
EPW = E // NW
CH = 80
NCHUNK = EPW // CH

@functools.cache
def _mesh():
    return plsc.VectorSubcoreMesh(core_axis_name="c", subcore_axis_name="s",
                                  num_cores=NC, num_subcores=NS)


def _ln(v, scale, bias):
    mu = jnp.mean(v, axis=-1, keepdims=True)
    var = jnp.mean((v - mu) ** 2, axis=-1, keepdims=True)
    return (v - mu) * lax.rsqrt(var + 1e-5) * scale + bias


def _mlp_body(x_ref, w1_ref, b1_ref, w2_ref, b2_ref, s_ref, t_ref, o_ref):
    u = jnp.maximum(
        jnp.dot(x_ref[...], w1_ref[...], preferred_element_type=jnp.float32)
        + b1_ref[...], 0.0)
    v = jnp.dot(u, w2_ref[...], preferred_element_type=jnp.float32) + b2_ref[...]
    o_ref[...] = _ln(v, s_ref[...], t_ref[...])


def _row2(a):
    return a.reshape(1, -1)


def _mlp(x, p, block_rows):
    (w1, b1), (w2, b2) = p["layers"]
    rows = x.shape[0]
    grid = (rows // block_rows,)
    full = lambda i: (0, 0)
    return pl.pallas_call(
        _mlp_body,
        grid=grid,
        in_specs=[
            pl.BlockSpec((block_rows, x.shape[1]), lambda i: (i, 0)),
            pl.BlockSpec(w1.shape, full),
            pl.BlockSpec((1, D), full),
            pl.BlockSpec(w2.shape, full),
            pl.BlockSpec((1, D), full),
            pl.BlockSpec((1, D), full),
            pl.BlockSpec((1, D), full),
        ],
        out_specs=pl.BlockSpec((block_rows, D), lambda i: (i, 0)),
        out_shape=jax.ShapeDtypeStruct((rows, D), jnp.float32),
    )(x, w1, _row2(b1), w2, _row2(b2), _row2(p["ln_scale"]), _row2(p["ln_bias"]))


def _edge_step_body(e_ref, g_ref, wc_ref, b1_ref,
                    w2_ref, b2_ref, s_ref, t_ref, enew_ref, eout_ref):
    e = e_ref[...]
    pre = (g_ref[...]
           + jnp.dot(e, wc_ref[...], preferred_element_type=jnp.float32)
           + b1_ref[...])
    u = jnp.maximum(pre, 0.0)
    v = jnp.dot(u, w2_ref[...], preferred_element_type=jnp.float32) + b2_ref[...]
    v = _ln(v, s_ref[...], t_ref[...])
    enew_ref[...] = v
    eout_ref[...] = e + v


def _edge_step(e, g, p, block_rows=1600):
    (w1, b1), (w2, b2) = p["layers"]
    wc = w1[2 * D:3 * D]
    grid = (E // block_rows,)
    full = lambda i: (0, 0)
    blk = lambda i: (i, 0)
    return pl.pallas_call(
        _edge_step_body,
        grid=grid,
        in_specs=[
            pl.BlockSpec((block_rows, D), blk),
            pl.BlockSpec((block_rows, D), blk),
            pl.BlockSpec((D, D), full),
            pl.BlockSpec((1, D), full),
            pl.BlockSpec((D, D), full),
            pl.BlockSpec((1, D), full),
            pl.BlockSpec((1, D), full),
            pl.BlockSpec((1, D), full),
        ],
        out_specs=(pl.BlockSpec((block_rows, D), blk),
                   pl.BlockSpec((block_rows, D), blk)),
        out_shape=(jax.ShapeDtypeStruct((E, D), jnp.float32),
                   jax.ShapeDtypeStruct((E, D), jnp.float32)),
    )(e, g, wc, _row2(b1), w2, _row2(b2),
      _row2(p["ln_scale"]), _row2(p["ln_bias"]))


def _mlp_rows(x, p_w1, p_b1, p_w2, p_b2, p_s, p_t):
    u = jnp.maximum(
        jnp.dot(x, p_w1, preferred_element_type=jnp.float32) + p_b1, 0.0)
    v = jnp.dot(u, p_w2, preferred_element_type=jnp.float32) + p_b2
    return _ln(v, p_s, p_t)


def _edge_step_first_body(ef_ref, g_ref, w1e, b1e, w2e, b2e, se, te,
                          wc, b1, w2, b2, s_, t_, enew_ref, eout_ref):
    e = _mlp_rows(ef_ref[...], w1e[...], b1e[...], w2e[...], b2e[...],
                  se[...], te[...])
    pre = (g_ref[...]
           + jnp.dot(e, wc[...], preferred_element_type=jnp.float32)
           + b1[...])
    u = jnp.maximum(pre, 0.0)
    v = jnp.dot(u, w2[...], preferred_element_type=jnp.float32) + b2[...]
    v = _ln(v, s_[...], t_[...])
    enew_ref[...] = v
    eout_ref[...] = e + v


def _edge_step_first(ef, g, pe, p, block_rows=1600):
    (w1e, b1e), (w2e, b2e) = pe["layers"]
    (w1, b1), (w2, b2) = p["layers"]
    wc = w1[2 * D:3 * D]
    grid = (E // block_rows,)
    full = lambda i: (0, 0)
    blk = lambda i: (i, 0)
    wspec = pl.BlockSpec((D, D), full)
    vspec = pl.BlockSpec((1, D), full)
    return pl.pallas_call(
        _edge_step_first_body,
        grid=grid,
        in_specs=[pl.BlockSpec((block_rows, D), blk),
                  pl.BlockSpec((block_rows, D), blk),
                  wspec, vspec, wspec, vspec, vspec, vspec,
                  wspec, vspec, wspec, vspec, vspec, vspec],
        out_specs=(pl.BlockSpec((block_rows, D), blk),
                   pl.BlockSpec((block_rows, D), blk)),
        out_shape=(jax.ShapeDtypeStruct((E, D), jnp.float32),
                   jax.ShapeDtypeStruct((E, D), jnp.float32)),
    )(ef, g, w1e, _row2(b1e), w2e, _row2(b2e),
      _row2(pe["ln_scale"]), _row2(pe["ln_bias"]),
      wc, _row2(b1), w2, _row2(b2),
      _row2(p["ln_scale"]), _row2(p["ln_bias"]))


def _edge_step_last_body(e_ref, g_ref, wc, b1, w2, b2, s_, t_,
                         w1d, b1d, w2d, b2d, sd, td, enew_ref, dec_ref):
    e = e_ref[...]
    pre = (g_ref[...]
           + jnp.dot(e, wc[...], preferred_element_type=jnp.float32)
           + b1[...])
    u = jnp.maximum(pre, 0.0)
    v = jnp.dot(u, w2[...], preferred_element_type=jnp.float32) + b2[...]
    v = _ln(v, s_[...], t_[...])
    enew_ref[...] = v
    dec_ref[...] = _mlp_rows(e + v, w1d[...], b1d[...], w2d[...], b2d[...],
                             sd[...], td[...])


def _edge_step_last(e, g, p, pd, block_rows=1600):
    (w1, b1), (w2, b2) = p["layers"]
    (w1d, b1d), (w2d, b2d) = pd["layers"]
    wc = w1[2 * D:3 * D]
    grid = (E // block_rows,)
    full = lambda i: (0, 0)
    blk = lambda i: (i, 0)
    wspec = pl.BlockSpec((D, D), full)
    vspec = pl.BlockSpec((1, D), full)
    return pl.pallas_call(
        _edge_step_last_body,
        grid=grid,
        in_specs=[pl.BlockSpec((block_rows, D), blk),
                  pl.BlockSpec((block_rows, D), blk),
                  wspec, vspec, wspec, vspec, vspec, vspec,
                  wspec, vspec, wspec, vspec, vspec, vspec],
        out_specs=(pl.BlockSpec((block_rows, D), blk),
                   pl.BlockSpec((block_rows, D), blk)),
        out_shape=(jax.ShapeDtypeStruct((E, D), jnp.float32),
                   jax.ShapeDtypeStruct((E, D), jnp.float32)),
    )(e, g, wc, _row2(b1), w2, _row2(b2),
      _row2(p["ln_scale"]), _row2(p["ln_bias"]),
      w1d, _row2(b1d), w2d, _row2(b2d),
      _row2(pd["ln_scale"]), _row2(pd["ln_bias"]))


def _node_step_body(h_ref, a0_ref, a1_ref, wh_ref, wg_ref, b1_ref, w2_ref,
                    b2_ref, s_ref, t_ref, o_ref):
    h = h_ref[...]
    agg = a0_ref[...] + a1_ref[...]
    u = jnp.maximum(
        jnp.dot(h, wh_ref[...], preferred_element_type=jnp.float32)
        + jnp.dot(agg, wg_ref[...], preferred_element_type=jnp.float32)
        + b1_ref[...], 0.0)
    v = jnp.dot(u, w2_ref[...], preferred_element_type=jnp.float32) + b2_ref[...]
    o_ref[...] = h + _ln(v, s_ref[...], t_ref[...])


def _node_step_tables_body(h_ref, a0_ref, a1_ref, wh_ref, wg_ref, b1_ref,
                           w2_ref, b2_ref, s_ref, t_ref, wa_ref, wb_ref,
                           o_ref, ha_ref, hb_ref):
    h = h_ref[...]
    agg = a0_ref[...] + a1_ref[...]
    u = jnp.maximum(
        jnp.dot(h, wh_ref[...], preferred_element_type=jnp.float32)
        + jnp.dot(agg, wg_ref[...], preferred_element_type=jnp.float32)
        + b1_ref[...], 0.0)
    v = jnp.dot(u, w2_ref[...], preferred_element_type=jnp.float32) + b2_ref[...]
    ho = h + _ln(v, s_ref[...], t_ref[...])
    o_ref[...] = ho
    ha_ref[...] = jnp.dot(ho, wa_ref[...], preferred_element_type=jnp.float32)
    hb_ref[...] = jnp.dot(ho, wb_ref[...], preferred_element_type=jnp.float32)


def _node_step(h, a0, a1, p, wa=None, wb=None, tables=False, block_rows=2000):
    (w1, b1), (w2, b2) = p["layers"]
    wh, wg = w1[0:D], w1[D:2 * D]
    grid = (N // block_rows,)
    full = lambda i: (0, 0)
    blk = lambda i: (i, 0)
    specs = [
        pl.BlockSpec((block_rows, D), blk),
        pl.BlockSpec((block_rows, D), blk),
        pl.BlockSpec((block_rows, D), blk),
        pl.BlockSpec((D, D), full),
        pl.BlockSpec((D, D), full),
        pl.BlockSpec((1, D), full),
        pl.BlockSpec((D, D), full),
        pl.BlockSpec((1, D), full),
        pl.BlockSpec((1, D), full),
        pl.BlockSpec((1, D), full),
    ]
    args = [h, a0, a1, wh, wg, _row2(b1), w2, _row2(b2),
            _row2(p["ln_scale"]), _row2(p["ln_bias"])]
    if not tables:
        return pl.pallas_call(
            _node_step_body,
            grid=grid,
            in_specs=specs,
            out_specs=pl.BlockSpec((block_rows, D), blk),
            out_shape=jax.ShapeDtypeStruct((N, D), jnp.float32),
        )(*args)
    specs += [pl.BlockSpec((D, D), full), pl.BlockSpec((D, D), full)]
    args += [wa, wb]
    return pl.pallas_call(
        _node_step_tables_body,
        grid=grid,
        in_specs=specs,
        out_specs=(pl.BlockSpec((block_rows, D), blk),) * 3,
        out_shape=(jax.ShapeDtypeStruct((N, D), jnp.float32),) * 3,
    )(*args)


def _tables_body(h_ref, wa_ref, wb_ref, ha_ref, hb_ref):
    h = h_ref[...]
    ha_ref[...] = jnp.dot(h, wa_ref[...], preferred_element_type=jnp.float32)
    hb_ref[...] = jnp.dot(h, wb_ref[...], preferred_element_type=jnp.float32)


def _tables(h, wa, wb, block_rows=2000):
    grid = (N // block_rows,)
    full = lambda i: (0, 0)
    blk = lambda i: (i, 0)
    return pl.pallas_call(
        _tables_body,
        grid=grid,
        in_specs=[
            pl.BlockSpec((block_rows, D), blk),
            pl.BlockSpec((D, D), full),
            pl.BlockSpec((D, D), full),
        ],
        out_specs=(pl.BlockSpec((block_rows, D), blk),) * 2,
        out_shape=(jax.ShapeDtypeStruct((N, D), jnp.float32),) * 2,
    )(h, wa, wb)


def _gather_body(ha_hbm, hb_hbm, src_hbm, dst_hbm, g_hbm,
                 sidx, didx,
                 ba0, ba1, ba2, bb0, bb1, bb2, wb0, wb1, wb2,
                 sa0, sa1, sa2, sb0, sb1, sb2, sw0, sw1, sw2):
    c = lax.axis_index("c")
    s = lax.axis_index("s")
    w = c * NS + s
    base = w * EPW
    pltpu.sync_copy(src_hbm.at[w], sidx)
    pltpu.sync_copy(dst_hbm.at[w], didx)

    slots = ((ba0, bb0, wb0, sa0, sb0, sw0),
             (ba1, bb1, wb1, sa1, sb1, sw1),
             (ba2, bb2, wb2, sa2, sb2, sw2))

    def out_ref(j):
        return g_hbm.at[pl.ds(pl.multiple_of(base + j * CH, CH), CH)]

    def start_g(j, t):
        ba, bb, _, sa, sb, _ = slots[t]
        pltpu.async_copy(ha_hbm.at[sidx.at[j]], ba, sa)
        pltpu.async_copy(hb_hbm.at[didx.at[j]], bb, sb)

    def visit(j, t, *, first, last):
        ba, bb, wb, sa, sb, sw = slots[t]
        pltpu.make_async_copy(ha_hbm.at[sidx.at[j]], ba, sa).wait()
        pltpu.make_async_copy(hb_hbm.at[didx.at[j]], bb, sb).wait()
        if not first:
            pltpu.make_async_copy(wb, out_ref(j - 3), sw).wait()

        def addrows(r4, carry):
            for r0 in range(4):
                r = r4 * 4 + r0
                for k in range(D // 16):
                    sl = pl.ds(k * 16, 16)
                    wb[r, sl] = ba[r, sl] + bb[r, sl]
            return carry

        lax.fori_loop(0, CH // 4, addrows, 0)
        if not last:
            start_g(j + 3, t)
        pltpu.async_copy(wb, out_ref(j), sw)

    start_g(0, 0)
    start_g(1, 1)
    start_g(2, 2)
    visit(0, 0, first=True, last=False)
    visit(1, 1, first=True, last=False)
    visit(2, 2, first=True, last=False)

    def body(i, carry):
        v0 = 3 * i + 3
        visit(v0, 0, first=False, last=False)
        visit(v0 + 1, 1, first=False, last=False)
        visit(v0 + 2, 2, first=False, last=False)
        return carry

    lax.fori_loop(0, 39, body, 0)
    visit(120, 0, first=False, last=False)
    visit(121, 1, first=False, last=False)
    visit(122, 2, first=False, last=True)
    visit(123, 0, first=False, last=True)
    visit(124, 1, first=False, last=True)
    for j, t in ((122, 2), (123, 0), (124, 1)):
        _, _, wb, _, _, sw = slots[t]
        pltpu.make_async_copy(wb, out_ref(j), sw).wait()


@functools.cache
def _sc_gather_kernel():
    return pl.kernel(
        _gather_body,
        out_type=jax.ShapeDtypeStruct((E, D), jnp.float32),
        mesh=_mesh(),
        scratch_types=(
            [pltpu.VMEM((NCHUNK, CH), jnp.int32)] * 2
            + [pltpu.VMEM((CH, D), jnp.float32)] * 9
            + [pltpu.SemaphoreType.DMA] * 9
        ),
    )


def _sc_gather(ha, hb, src_r, dst_r):
    return _sc_gather_kernel()(ha, hb, src_r, dst_r)


ZROWS = 632
ZLAST = N - (NS - 1) * ZROWS
ZB = 8


def _scatter_body(enew_hbm, dst_hbm, agg_hbm, didx,
                  b0, b1, b2, zbuf, acc,
                  sf0, sf1, sf2, sa0, sa1, sa2, semz):
    c = lax.axis_index("c")
    s = lax.axis_index("s")
    w = c * NS + s

    z16 = jnp.zeros((16,), jnp.float32)
    for i in range(ZB):
        for k in range(D // 16):
            zbuf[i, pl.ds(k * 16, 16)] = z16

    my_base = pl.multiple_of(s * ZROWS, ZROWS)
    my_rows = jnp.where(s == NS - 1, ZLAST, ZROWS)

    def zrow(r, carry):
        pltpu.async_copy(
            zbuf, acc.at[pl.ds(pl.multiple_of(my_base + r * ZB, ZB), ZB)], semz)
        return carry

    def zdrain(r, carry):
        pltpu.make_async_copy(
            zbuf, acc.at[pl.ds(pl.multiple_of(my_base + r * ZB, ZB), ZB)],
            semz).wait()
        return carry

    nz = my_rows // ZB
    lax.fori_loop(0, nz, zrow, 0)
    lax.fori_loop(0, nz, zdrain, 0)
    plsc.subcore_barrier()

    pltpu.sync_copy(dst_hbm.at[w], didx)

    slots = ((b0, sf0, sa0), (b1, sf1, sa1), (b2, sf2, sa2))

    def in_ref(j):
        return enew_hbm.at[pl.ds(pl.multiple_of(w * EPW + j * CH, CH), CH)]

    def fetch(j, t):
        b, sf, _ = slots[t]
        pltpu.async_copy(in_ref(j), b, sf)

    def wait_add(j, t):
        b, _, sa = slots[t]
        pltpu.make_async_copy(b, acc.at[didx.at[j]], sa).wait()

    def sync_add(j, t):
        b, sf, sa = slots[t]
        pltpu.make_async_copy(in_ref(j), b, sf).wait()
        pltpu.sync_copy(b, acc.at[didx.at[j]], add=True)

    fetch(0, 0)
    fetch(1, 1)
    fetch(2, 2)

    def body(i, carry):
        v0 = 3 * i
        sync_add(v0, 0)
        fetch(v0 + 3, 0)
        sync_add(v0 + 1, 1)
        fetch(v0 + 4, 1)
        sync_add(v0 + 2, 2)
        fetch(v0 + 5, 2)
        return carry

    lax.fori_loop(0, (NCHUNK - 5) // 3, body, 0)
    sync_add(120, 0)
    fetch(123, 0)
    sync_add(121, 1)
    fetch(124, 1)
    sync_add(122, 2)
    sync_add(123, 0)
    sync_add(124, 1)
    plsc.subcore_barrier()

    @pl.when(s < NS - 1)
    def _():
        pltpu.async_copy(acc.at[pl.ds(my_base, ZROWS)],
                         agg_hbm.at[c, pl.ds(my_base, ZROWS)], semz).wait()

    @pl.when(s == NS - 1)
    def _():
        pltpu.async_copy(acc.at[pl.ds((NS - 1) * ZROWS, ZLAST)],
                         agg_hbm.at[c, pl.ds((NS - 1) * ZROWS, ZLAST)],
                         semz).wait()


@functools.cache
def _sc_scatter_kernel():
    return pl.kernel(
        _scatter_body,
        out_type=jax.ShapeDtypeStruct((NC, N, D), jnp.float32),
        mesh=_mesh(),
        scratch_types=(
            [pltpu.VMEM((NCHUNK, CH), jnp.int32)]
            + [pltpu.VMEM((CH, D), jnp.float32)] * 3
            + [pltpu.VMEM((ZB, D), jnp.float32)]
            + [pltpu.VMEM_SHARED((N, D), jnp.float32)]
            + [pltpu.SemaphoreType.DMA] * 7
        ),
    )


def _sc_scatter(e_new, dst_r):
    return _sc_scatter_kernel()(e_new, dst_r)


def kernel(x, edge_index, edge_features, params):
    src_r = edge_index[0].reshape(NW, NCHUNK, CH)
    dst_r = edge_index[1].reshape(NW, NCHUNK, CH)

    h = _mlp(x, params["enc_node"], block_rows=2000)

    wa0 = params["proc"][0]["edge"]["layers"][0][0][0:D]
    wb0 = params["proc"][0]["edge"]["layers"][0][0][D:2 * D]
    ha, hb = _tables(h, wa0, wb0)
    nsteps = len(params["proc"])
    e = None
    for i, p in enumerate(params["proc"]):
        g = _sc_gather(ha, hb, src_r, dst_r)
        if i == 0:
            e_new, e = _edge_step_first(edge_features, g,
                                        params["enc_edge"], p["edge"])
        elif i == nsteps - 1:
            e_new, dec_e = _edge_step_last(e, g, p["edge"],
                                           params["dec_edge"])
        else:
            e_new, e = _edge_step(e, g, p["edge"])
        agg = _sc_scatter(e_new, dst_r)
        if i + 1 < nsteps:
            wan = params["proc"][i + 1]["edge"]["layers"][0][0][0:D]
            wbn = params["proc"][i + 1]["edge"]["layers"][0][0][D:2 * D]
            h, ha, hb = _node_step(h, agg[0], agg[1], p["node"],
                                   wan, wbn, tables=True)
        else:
            h = _node_step(h, agg[0], agg[1], p["node"])

    return (_mlp(h, params["dec_node"], block_rows=2000), dec_e)


# speedup vs baseline: 1.1726x; 1.0481x over previous
import functools

import jax
import jax.numpy as jnp
from jax import lax
from jax.experimental import pallas as pl
from jax.experimental.pallas import tpu as pltpu
from jax.experimental.pallas import tpu_sc as plsc

N = 10000
E = 320000
D = 128

NC = 2
NS = 16
NW = NC * NS
EPW = E // NW
CH = 80
NCHUNK = EPW // CH

@functools.cache
def _mesh():
    return plsc.VectorSubcoreMesh(core_axis_name="c", subcore_axis_name="s",
                                  num_cores=NC, num_subcores=NS)


def _ln(v, scale, bias):
    mu = jnp.mean(v, axis=-1, keepdims=True)
    var = jnp.mean((v - mu) ** 2, axis=-1, keepdims=True)
    return (v - mu) * lax.rsqrt(var + 1e-5) * scale + bias


def _mlp_body(x_ref, w1_ref, b1_ref, w2_ref, b2_ref, s_ref, t_ref, o_ref):
    u = jnp.maximum(
        jnp.dot(x_ref[...], w1_ref[...], preferred_element_type=jnp.float32)
        + b1_ref[...], 0.0)
    v = jnp.dot(u, w2_ref[...], preferred_element_type=jnp.float32) + b2_ref[...]
    o_ref[...] = _ln(v, s_ref[...], t_ref[...])


def _row2(a):
    return a.reshape(1, -1)


def _mlp(x, p, block_rows):
    (w1, b1), (w2, b2) = p["layers"]
    rows = x.shape[0]
    grid = (rows // block_rows,)
    full = lambda i: (0, 0)
    return pl.pallas_call(
        _mlp_body,
        grid=grid,
        in_specs=[
            pl.BlockSpec((block_rows, x.shape[1]), lambda i: (i, 0)),
            pl.BlockSpec(w1.shape, full),
            pl.BlockSpec((1, D), full),
            pl.BlockSpec(w2.shape, full),
            pl.BlockSpec((1, D), full),
            pl.BlockSpec((1, D), full),
            pl.BlockSpec((1, D), full),
        ],
        out_specs=pl.BlockSpec((block_rows, D), lambda i: (i, 0)),
        out_shape=jax.ShapeDtypeStruct((rows, D), jnp.float32),
    )(x, w1, _row2(b1), w2, _row2(b2), _row2(p["ln_scale"]), _row2(p["ln_bias"]))


def _edge_step_body(e_ref, g_ref, wc_ref, b1_ref,
                    w2_ref, b2_ref, s_ref, t_ref, enew_ref, eout_ref):
    e = e_ref[...]
    pre = (g_ref[...]
           + jnp.dot(e, wc_ref[...], preferred_element_type=jnp.float32)
           + b1_ref[...])
    u = jnp.maximum(pre, 0.0)
    v = jnp.dot(u, w2_ref[...], preferred_element_type=jnp.float32) + b2_ref[...]
    v = _ln(v, s_ref[...], t_ref[...])
    enew_ref[...] = v
    eout_ref[...] = e + v


def _edge_step(e, g, p, block_rows=1600):
    (w1, b1), (w2, b2) = p["layers"]
    wc = w1[2 * D:3 * D]
    rows = e.shape[0]
    grid = (rows // block_rows,)
    full = lambda i: (0, 0)
    blk = lambda i: (i, 0)
    return pl.pallas_call(
        _edge_step_body,
        grid=grid,
        in_specs=[
            pl.BlockSpec((block_rows, D), blk),
            pl.BlockSpec((block_rows, D), blk),
            pl.BlockSpec((D, D), full),
            pl.BlockSpec((1, D), full),
            pl.BlockSpec((D, D), full),
            pl.BlockSpec((1, D), full),
            pl.BlockSpec((1, D), full),
            pl.BlockSpec((1, D), full),
        ],
        out_specs=(pl.BlockSpec((block_rows, D), blk),
                   pl.BlockSpec((block_rows, D), blk)),
        out_shape=(jax.ShapeDtypeStruct((rows, D), jnp.float32),
                   jax.ShapeDtypeStruct((rows, D), jnp.float32)),
    )(e, g, wc, _row2(b1), w2, _row2(b2),
      _row2(p["ln_scale"]), _row2(p["ln_bias"]))


def _mlp_rows(x, p_w1, p_b1, p_w2, p_b2, p_s, p_t):
    u = jnp.maximum(
        jnp.dot(x, p_w1, preferred_element_type=jnp.float32) + p_b1, 0.0)
    v = jnp.dot(u, p_w2, preferred_element_type=jnp.float32) + p_b2
    return _ln(v, p_s, p_t)


def _edge_step_first_body(ef_ref, g_ref, w1e, b1e, w2e, b2e, se, te,
                          wc, b1, w2, b2, s_, t_, enew_ref, eout_ref):
    e = _mlp_rows(ef_ref[...], w1e[...], b1e[...], w2e[...], b2e[...],
                  se[...], te[...])
    pre = (g_ref[...]
           + jnp.dot(e, wc[...], preferred_element_type=jnp.float32)
           + b1[...])
    u = jnp.maximum(pre, 0.0)
    v = jnp.dot(u, w2[...], preferred_element_type=jnp.float32) + b2[...]
    v = _ln(v, s_[...], t_[...])
    enew_ref[...] = v
    eout_ref[...] = e + v


def _edge_step_first(ef, g, pe, p, block_rows=1600):
    (w1e, b1e), (w2e, b2e) = pe["layers"]
    (w1, b1), (w2, b2) = p["layers"]
    wc = w1[2 * D:3 * D]
    rows = ef.shape[0]
    grid = (rows // block_rows,)
    full = lambda i: (0, 0)
    blk = lambda i: (i, 0)
    wspec = pl.BlockSpec((D, D), full)
    vspec = pl.BlockSpec((1, D), full)
    return pl.pallas_call(
        _edge_step_first_body,
        grid=grid,
        in_specs=[pl.BlockSpec((block_rows, D), blk),
                  pl.BlockSpec((block_rows, D), blk),
                  wspec, vspec, wspec, vspec, vspec, vspec,
                  wspec, vspec, wspec, vspec, vspec, vspec],
        out_specs=(pl.BlockSpec((block_rows, D), blk),
                   pl.BlockSpec((block_rows, D), blk)),
        out_shape=(jax.ShapeDtypeStruct((rows, D), jnp.float32),
                   jax.ShapeDtypeStruct((rows, D), jnp.float32)),
    )(ef, g, w1e, _row2(b1e), w2e, _row2(b2e),
      _row2(pe["ln_scale"]), _row2(pe["ln_bias"]),
      wc, _row2(b1), w2, _row2(b2),
      _row2(p["ln_scale"]), _row2(p["ln_bias"]))


def _edge_step_last_body(e_ref, g_ref, wc, b1, w2, b2, s_, t_,
                         w1d, b1d, w2d, b2d, sd, td, enew_ref, dec_ref):
    e = e_ref[...]
    pre = (g_ref[...]
           + jnp.dot(e, wc[...], preferred_element_type=jnp.float32)
           + b1[...])
    u = jnp.maximum(pre, 0.0)
    v = jnp.dot(u, w2[...], preferred_element_type=jnp.float32) + b2[...]
    v = _ln(v, s_[...], t_[...])
    enew_ref[...] = v
    dec_ref[...] = _mlp_rows(e + v, w1d[...], b1d[...], w2d[...], b2d[...],
                             sd[...], td[...])


def _edge_step_last(e, g, p, pd, block_rows=1600):
    (w1, b1), (w2, b2) = p["layers"]
    (w1d, b1d), (w2d, b2d) = pd["layers"]
    wc = w1[2 * D:3 * D]
    rows = e.shape[0]
    grid = (rows // block_rows,)
    full = lambda i: (0, 0)
    blk = lambda i: (i, 0)
    wspec = pl.BlockSpec((D, D), full)
    vspec = pl.BlockSpec((1, D), full)
    return pl.pallas_call(
        _edge_step_last_body,
        grid=grid,
        in_specs=[pl.BlockSpec((block_rows, D), blk),
                  pl.BlockSpec((block_rows, D), blk),
                  wspec, vspec, wspec, vspec, vspec, vspec,
                  wspec, vspec, wspec, vspec, vspec, vspec],
        out_specs=(pl.BlockSpec((block_rows, D), blk),
                   pl.BlockSpec((block_rows, D), blk)),
        out_shape=(jax.ShapeDtypeStruct((rows, D), jnp.float32),
                   jax.ShapeDtypeStruct((rows, D), jnp.float32)),
    )(e, g, wc, _row2(b1), w2, _row2(b2),
      _row2(p["ln_scale"]), _row2(p["ln_bias"]),
      w1d, _row2(b1d), w2d, _row2(b2d),
      _row2(pd["ln_scale"]), _row2(pd["ln_bias"]))


def _node_step_body(h_ref, a0_ref, a1_ref, a2_ref, a3_ref, wh_ref, wg_ref,
                    b1_ref, w2_ref, b2_ref, s_ref, t_ref, o_ref):
    h = h_ref[...]
    agg = (a0_ref[...] + a1_ref[...]) + (a2_ref[...] + a3_ref[...])
    u = jnp.maximum(
        jnp.dot(h, wh_ref[...], preferred_element_type=jnp.float32)
        + jnp.dot(agg, wg_ref[...], preferred_element_type=jnp.float32)
        + b1_ref[...], 0.0)
    v = jnp.dot(u, w2_ref[...], preferred_element_type=jnp.float32) + b2_ref[...]
    o_ref[...] = h + _ln(v, s_ref[...], t_ref[...])


def _node_step_tables_body(h_ref, a0_ref, a1_ref, a2_ref, a3_ref, wh_ref,
                           wg_ref, b1_ref, w2_ref, b2_ref, s_ref, t_ref,
                           wa_ref, wb_ref, o_ref, ha_ref, hb_ref):
    h = h_ref[...]
    agg = (a0_ref[...] + a1_ref[...]) + (a2_ref[...] + a3_ref[...])
    u = jnp.maximum(
        jnp.dot(h, wh_ref[...], preferred_element_type=jnp.float32)
        + jnp.dot(agg, wg_ref[...], preferred_element_type=jnp.float32)
        + b1_ref[...], 0.0)
    v = jnp.dot(u, w2_ref[...], preferred_element_type=jnp.float32) + b2_ref[...]
    ho = h + _ln(v, s_ref[...], t_ref[...])
    o_ref[...] = ho
    ha_ref[...] = jnp.dot(ho, wa_ref[...], preferred_element_type=jnp.float32)
    hb_ref[...] = jnp.dot(ho, wb_ref[...], preferred_element_type=jnp.float32)


def _node_step(h, a0, a1, a2, a3, p, wa=None, wb=None, tables=False,
               block_rows=2000):
    (w1, b1), (w2, b2) = p["layers"]
    wh, wg = w1[0:D], w1[D:2 * D]
    grid = (N // block_rows,)
    full = lambda i: (0, 0)
    blk = lambda i: (i, 0)
    specs = [
        pl.BlockSpec((block_rows, D), blk),
        pl.BlockSpec((block_rows, D), blk),
        pl.BlockSpec((block_rows, D), blk),
        pl.BlockSpec((block_rows, D), blk),
        pl.BlockSpec((block_rows, D), blk),
        pl.BlockSpec((D, D), full),
        pl.BlockSpec((D, D), full),
        pl.BlockSpec((1, D), full),
        pl.BlockSpec((D, D), full),
        pl.BlockSpec((1, D), full),
        pl.BlockSpec((1, D), full),
        pl.BlockSpec((1, D), full),
    ]
    args = [h, a0, a1, a2, a3, wh, wg, _row2(b1), w2, _row2(b2),
            _row2(p["ln_scale"]), _row2(p["ln_bias"])]
    if not tables:
        return pl.pallas_call(
            _node_step_body,
            grid=grid,
            in_specs=specs,
            out_specs=pl.BlockSpec((block_rows, D), blk),
            out_shape=jax.ShapeDtypeStruct((N, D), jnp.float32),
        )(*args)
    specs += [pl.BlockSpec((D, D), full), pl.BlockSpec((D, D), full)]
    args += [wa, wb]
    return pl.pallas_call(
        _node_step_tables_body,
        grid=grid,
        in_specs=specs,
        out_specs=(pl.BlockSpec((block_rows, D), blk),) * 3,
        out_shape=(jax.ShapeDtypeStruct((N, D), jnp.float32),) * 3,
    )(*args)


def _tables_body(h_ref, wa_ref, wb_ref, ha_ref, hb_ref):
    h = h_ref[...]
    ha_ref[...] = jnp.dot(h, wa_ref[...], preferred_element_type=jnp.float32)
    hb_ref[...] = jnp.dot(h, wb_ref[...], preferred_element_type=jnp.float32)


def _tables(h, wa, wb, block_rows=2000):
    grid = (N // block_rows,)
    full = lambda i: (0, 0)
    blk = lambda i: (i, 0)
    return pl.pallas_call(
        _tables_body,
        grid=grid,
        in_specs=[
            pl.BlockSpec((block_rows, D), blk),
            pl.BlockSpec((D, D), full),
            pl.BlockSpec((D, D), full),
        ],
        out_specs=(pl.BlockSpec((block_rows, D), blk),) * 2,
        out_shape=(jax.ShapeDtypeStruct((N, D), jnp.float32),) * 2,
    )(h, wa, wb)


def _make_gather_body(epw, ch):
    def _gather_body(ha_hbm, hb_hbm, src_hbm, dst_hbm, g_hbm,
                     sidx, didx,
                     ba0, ba1, ba2, bb0, bb1, bb2, wb0, wb1, wb2,
                     sa0, sa1, sa2, sb0, sb1, sb2, sw0, sw1, sw2):
        c = lax.axis_index("c")
        s = lax.axis_index("s")
        w = c * NS + s
        base = w * epw
        pltpu.sync_copy(src_hbm.at[w], sidx)
        pltpu.sync_copy(dst_hbm.at[w], didx)

        slots = ((ba0, bb0, wb0, sa0, sb0, sw0),
                 (ba1, bb1, wb1, sa1, sb1, sw1),
                 (ba2, bb2, wb2, sa2, sb2, sw2))

        def out_ref(j):
            return g_hbm.at[pl.ds(pl.multiple_of(base + j * ch, ch), ch)]

        def start_g(j, t):
            ba, bb, _, sa, sb, _ = slots[t]
            pltpu.async_copy(ha_hbm.at[sidx.at[j]], ba, sa)
            pltpu.async_copy(hb_hbm.at[didx.at[j]], bb, sb)

        def visit(j, t, *, first, last):
            ba, bb, wb, sa, sb, sw = slots[t]
            pltpu.make_async_copy(ha_hbm.at[sidx.at[j]], ba, sa).wait()
            pltpu.make_async_copy(hb_hbm.at[didx.at[j]], bb, sb).wait()
            if not first:
                pltpu.make_async_copy(wb, out_ref(j - 3), sw).wait()

            def addrows(r4, carry):
                for r0 in range(4):
                    r = r4 * 4 + r0
                    for k in range(D // 16):
                        sl = pl.ds(k * 16, 16)
                        wb[r, sl] = ba[r, sl] + bb[r, sl]
                return carry

            lax.fori_loop(0, ch // 4, addrows, 0)
            if not last:
                start_g(j + 3, t)
            pltpu.async_copy(wb, out_ref(j), sw)

        start_g(0, 0)
        start_g(1, 1)
        start_g(2, 2)
        visit(0, 0, first=True, last=False)
        visit(1, 1, first=True, last=False)
        visit(2, 2, first=True, last=False)

        def body(i, carry):
            v0 = 3 * i + 3
            visit(v0, 0, first=False, last=False)
            visit(v0 + 1, 1, first=False, last=False)
            visit(v0 + 2, 2, first=False, last=False)
            return carry

        lax.fori_loop(0, 39, body, 0)
        visit(120, 0, first=False, last=False)
        visit(121, 1, first=False, last=False)
        visit(122, 2, first=False, last=True)
        visit(123, 0, first=False, last=True)
        visit(124, 1, first=False, last=True)
        for j, t in ((122, 2), (123, 0), (124, 1)):
            _, _, wb, _, _, sw = slots[t]
            pltpu.make_async_copy(wb, out_ref(j), sw).wait()

    return _gather_body


@functools.cache
def _sc_gather_kernel(epw, ch):
    edges = epw * NW
    return pl.kernel(
        _make_gather_body(epw, ch),
        out_type=jax.ShapeDtypeStruct((edges, D), jnp.float32),
        mesh=_mesh(),
        scratch_types=(
            [pltpu.VMEM((NCHUNK, ch), jnp.int32)] * 2
            + [pltpu.VMEM((ch, D), jnp.float32)] * 9
            + [pltpu.SemaphoreType.DMA] * 9
        ),
    )


def _sc_gather(ha, hb, src_r, dst_r):
    epw = src_r.shape[1] * src_r.shape[2]
    return _sc_gather_kernel(epw, src_r.shape[2])(ha, hb, src_r, dst_r)


ZROWS = 632
ZLAST = N - (NS - 1) * ZROWS
ZB = 8


def _make_scatter_body(epw, ch):
    def _scatter_body(enew_hbm, dst_hbm, agg_hbm, didx,
                      b0, b1, b2, zbuf, acc,
                      sf0, sf1, sf2, sa0, sa1, sa2, semz):
        c = lax.axis_index("c")
        s = lax.axis_index("s")
        w = c * NS + s

        z16 = jnp.zeros((16,), jnp.float32)
        for i in range(ZB):
            for k in range(D // 16):
                zbuf[i, pl.ds(k * 16, 16)] = z16

        my_base = pl.multiple_of(s * ZROWS, ZROWS)
        my_rows = jnp.where(s == NS - 1, ZLAST, ZROWS)

        def zrow(r, carry):
            pltpu.async_copy(
                zbuf, acc.at[pl.ds(pl.multiple_of(my_base + r * ZB, ZB), ZB)],
                semz)
            return carry

        def zdrain(r, carry):
            pltpu.make_async_copy(
                zbuf, acc.at[pl.ds(pl.multiple_of(my_base + r * ZB, ZB), ZB)],
                semz).wait()
            return carry

        nz = my_rows // ZB
        lax.fori_loop(0, nz, zrow, 0)
        lax.fori_loop(0, nz, zdrain, 0)
        plsc.subcore_barrier()

        pltpu.sync_copy(dst_hbm.at[w], didx)

        slots = ((b0, sf0, sa0), (b1, sf1, sa1), (b2, sf2, sa2))

        def in_ref(j):
            return enew_hbm.at[pl.ds(pl.multiple_of(w * epw + j * ch, ch), ch)]

        def fetch(j, t):
            b, sf, _ = slots[t]
            pltpu.async_copy(in_ref(j), b, sf)

        def sync_add(j, t):
            b, sf, sa = slots[t]
            pltpu.make_async_copy(in_ref(j), b, sf).wait()
            pltpu.sync_copy(b, acc.at[didx.at[j]], add=True)

        fetch(0, 0)
        fetch(1, 1)
        fetch(2, 2)

        def body(i, carry):
            v0 = 3 * i
            sync_add(v0, 0)
            fetch(v0 + 3, 0)
            sync_add(v0 + 1, 1)
            fetch(v0 + 4, 1)
            sync_add(v0 + 2, 2)
            fetch(v0 + 5, 2)
            return carry

        lax.fori_loop(0, (NCHUNK - 5) // 3, body, 0)
        sync_add(120, 0)
        fetch(123, 0)
        sync_add(121, 1)
        fetch(124, 1)
        sync_add(122, 2)
        sync_add(123, 0)
        sync_add(124, 1)
        plsc.subcore_barrier()

        @pl.when(s < NS - 1)
        def _():
            pltpu.async_copy(acc.at[pl.ds(my_base, ZROWS)],
                             agg_hbm.at[c, pl.ds(my_base, ZROWS)], semz).wait()

        @pl.when(s == NS - 1)
        def _():
            pltpu.async_copy(acc.at[pl.ds((NS - 1) * ZROWS, ZLAST)],
                             agg_hbm.at[c, pl.ds((NS - 1) * ZROWS, ZLAST)],
                             semz).wait()

    return _scatter_body


@functools.cache
def _sc_scatter_kernel(epw, ch):
    return pl.kernel(
        _make_scatter_body(epw, ch),
        out_type=jax.ShapeDtypeStruct((NC, N, D), jnp.float32),
        mesh=_mesh(),
        scratch_types=(
            [pltpu.VMEM((NCHUNK, ch), jnp.int32)]
            + [pltpu.VMEM((ch, D), jnp.float32)] * 3
            + [pltpu.VMEM((ZB, D), jnp.float32)]
            + [pltpu.VMEM_SHARED((N, D), jnp.float32)]
            + [pltpu.SemaphoreType.DMA] * 7
        ),
    )


def _sc_scatter(e_new, dst_r):
    epw = dst_r.shape[1] * dst_r.shape[2]
    return _sc_scatter_kernel(epw, dst_r.shape[2])(e_new, dst_r)


def kernel(x, edge_index, edge_features, params):
    EH = E // 2
    CH2 = 40
    src_a = edge_index[0, :EH].reshape(NW, NCHUNK, CH2)
    dst_a = edge_index[1, :EH].reshape(NW, NCHUNK, CH2)
    src_b = edge_index[0, EH:].reshape(NW, NCHUNK, CH2)
    dst_b = edge_index[1, EH:].reshape(NW, NCHUNK, CH2)
    ef_a = edge_features[:EH]
    ef_b = edge_features[EH:]

    h = _mlp(x, params["enc_node"], block_rows=2000)

    wa0 = params["proc"][0]["edge"]["layers"][0][0][0:D]
    wb0 = params["proc"][0]["edge"]["layers"][0][0][D:2 * D]
    ha, hb = _tables(h, wa0, wb0)
    nsteps = len(params["proc"])
    e_a = e_b = None
    for i, p in enumerate(params["proc"]):
        g_a = _sc_gather(ha, hb, src_a, dst_a)
        if i == 0:
            en_a, e_a = _edge_step_first(ef_a, g_a, params["enc_edge"],
                                         p["edge"])
            g_b = _sc_gather(ha, hb, src_b, dst_b)
            en_b, e_b = _edge_step_first(ef_b, g_b, params["enc_edge"],
                                         p["edge"])
        elif i == nsteps - 1:
            en_a, dec_a = _edge_step_last(e_a, g_a, p["edge"],
                                          params["dec_edge"])
            g_b = _sc_gather(ha, hb, src_b, dst_b)
            en_b, dec_b = _edge_step_last(e_b, g_b, p["edge"],
                                          params["dec_edge"])
        else:
            en_a, e_a = _edge_step(e_a, g_a, p["edge"])
            g_b = _sc_gather(ha, hb, src_b, dst_b)
            en_b, e_b = _edge_step(e_b, g_b, p["edge"])
        agg_a = _sc_scatter(en_a, dst_a)
        agg_b = _sc_scatter(en_b, dst_b)
        if i + 1 < nsteps:
            wan = params["proc"][i + 1]["edge"]["layers"][0][0][0:D]
            wbn = params["proc"][i + 1]["edge"]["layers"][0][0][D:2 * D]
            h, ha, hb = _node_step(h, agg_a[0], agg_a[1], agg_b[0], agg_b[1],
                                   p["node"], wan, wbn, tables=True)
        else:
            h = _node_step(h, agg_a[0], agg_a[1], agg_b[0], agg_b[1],
                           p["node"])

    return (_mlp(h, params["dec_node"], block_rows=2000),
            jnp.concatenate([dec_a, dec_b], axis=0))
